# Initial kernel scaffold; baseline (speedup 1.0000x reference)
import jax
import jax.numpy as jnp
from jax.experimental import pallas as pl


def kernel(indices, emb_table):
    raise NotImplementedError("write your pallas kernel here")


# speedup vs baseline: 4.5947x; 4.5947x over previous
import functools

import jax
import jax.numpy as jnp
from jax import lax
from jax.experimental import pallas as pl
from jax.experimental.pallas import tpu as pltpu
from jax.experimental.pallas import tpu_sc as plsc

_BATCH = 4096
_HIST = 50
_DIM = 64
_N = _BATCH * _HIST
_NC = 2
_NS = 16
_NW = _NC * _NS
_BPW = _N // _NW
_CHUNK = 800


def kernel(indices, emb_table):
    flat_idx = indices.reshape(_N).astype(jnp.int32)
    mesh = plsc.VectorSubcoreMesh(core_axis_name="c", subcore_axis_name="s")

    @functools.partial(
        pl.kernel,
        mesh=mesh,
        out_type=jax.ShapeDtypeStruct((_N, _DIM), jnp.float32),
        compiler_params=pltpu.CompilerParams(use_tc_tiling_on_sc=False),
        scratch_types=[
            pltpu.VMEM((_BPW,), jnp.int32),
            pltpu.VMEM((_CHUNK, _DIM), jnp.float32),
            pltpu.SemaphoreType.DMA,
        ],
    )
    def gather_kernel(table_hbm, idx_hbm, out_hbm, idx_v, rows_v, sem):
        wid = lax.axis_index("s") * _NC + lax.axis_index("c")
        base = wid * _BPW
        pltpu.sync_copy(idx_hbm.at[pl.ds(base, _BPW)], idx_v)

        @pl.loop(0, _BPW, step=_CHUNK)
        def _(off):
            pltpu.async_copy(
                table_hbm.at[idx_v.at[pl.ds(off, _CHUNK)]], rows_v, sem
            ).wait()
            pltpu.sync_copy(rows_v, out_hbm.at[pl.ds(base + off, _CHUNK)])

    out = gather_kernel(emb_table, flat_idx)
    return out.reshape(_BATCH, _HIST, _DIM)


# speedup vs baseline: 4.6138x; 1.0042x over previous
import functools

import jax
import jax.numpy as jnp
from jax import lax
from jax.experimental import pallas as pl
from jax.experimental.pallas import tpu as pltpu
from jax.experimental.pallas import tpu_sc as plsc

_BATCH = 4096
_HIST = 50
_DIM = 64
_N = _BATCH * _HIST
_NC = 2
_NS = 16
_NW = _NC * _NS
_BPW = _N // _NW
_CHUNK = 800


def kernel(indices, emb_table):
    flat_idx = indices.reshape(_N).astype(jnp.int32)
    mesh = plsc.VectorSubcoreMesh(core_axis_name="c", subcore_axis_name="s")

    @functools.partial(
        pl.kernel,
        mesh=mesh,
        out_type=jax.ShapeDtypeStruct((_N, _DIM), jnp.float32),
        compiler_params=pltpu.CompilerParams(use_tc_tiling_on_sc=False),
        scratch_types=[
            pltpu.VMEM((_BPW,), jnp.int32),
            pltpu.VMEM((_CHUNK, _DIM), jnp.float32),
            pltpu.VMEM((_CHUNK, _DIM), jnp.float32),
            pltpu.SemaphoreType.DMA,
            pltpu.SemaphoreType.DMA,
            pltpu.SemaphoreType.DMA,
            pltpu.SemaphoreType.DMA,
        ],
    )
    def gather_kernel(
        table_hbm, idx_hbm, out_hbm, idx_v, rows0, rows1, g0, g1, w0, w1
    ):
        wid = lax.axis_index("s") * _NC + lax.axis_index("c")
        base = wid * _BPW
        pltpu.sync_copy(idx_hbm.at[pl.ds(base, _BPW)], idx_v)

        rows = [rows0, rows1]
        gsem = [g0, g1]
        wsem = [w0, w1]
        n_chunks = _BPW // _CHUNK

        def gather_chunk(c):
            return pltpu.async_copy(
                table_hbm.at[idx_v.at[pl.ds(c * _CHUNK, _CHUNK)]],
                rows[c % 2],
                gsem[c % 2],
            )

        def write_chunk(c):
            return pltpu.async_copy(
                rows[c % 2],
                out_hbm.at[pl.ds(base + c * _CHUNK, _CHUNK)],
                wsem[c % 2],
            )

        gathers = {0: gather_chunk(0)}
        writes = {}
        for c in range(n_chunks):
            gathers[c].wait()
            if c + 1 < n_chunks:
                if c - 1 >= 0:
                    writes[c - 1].wait()
                gathers[c + 1] = gather_chunk(c + 1)
            writes[c] = write_chunk(c)
        writes[n_chunks - 1].wait()

    out = gather_kernel(emb_table, flat_idx)
    return out.reshape(_BATCH, _HIST, _DIM)


# speedup vs baseline: 4.6653x; 1.0112x over previous
import functools

import jax
import jax.numpy as jnp
from jax import lax
from jax.experimental import pallas as pl
from jax.experimental.pallas import tpu as pltpu
from jax.experimental.pallas import tpu_sc as plsc

_BATCH = 4096
_HIST = 50
_DIM = 64
_N = _BATCH * _HIST
_NC = 2
_NS = 16
_NW = _NC * _NS
_BPW = _N // _NW
_CHUNK = 400
_NBUF = 4


def kernel(indices, emb_table):
    flat_idx = indices.reshape(_N).astype(jnp.int32)
    mesh = plsc.VectorSubcoreMesh(core_axis_name="c", subcore_axis_name="s")

    @functools.partial(
        pl.kernel,
        mesh=mesh,
        out_type=jax.ShapeDtypeStruct((_N, _DIM), jnp.float32),
        compiler_params=pltpu.CompilerParams(use_tc_tiling_on_sc=False),
        scratch_types=(
            [pltpu.VMEM((_BPW,), jnp.int32)]
            + [pltpu.VMEM((_CHUNK, _DIM), jnp.float32)] * _NBUF
            + [pltpu.SemaphoreType.DMA] * (2 * _NBUF)
        ),
    )
    def gather_kernel(table_hbm, idx_hbm, out_hbm, idx_v, *bufs):
        rows = bufs[:_NBUF]
        gsem = bufs[_NBUF : 2 * _NBUF]
        wsem = bufs[2 * _NBUF :]
        wid = lax.axis_index("s") * _NC + lax.axis_index("c")
        base = wid * _BPW
        pltpu.sync_copy(idx_hbm.at[pl.ds(base, _BPW)], idx_v)

        n_chunks = _BPW // _CHUNK

        def gather_chunk(c):
            return pltpu.async_copy(
                table_hbm.at[idx_v.at[pl.ds(c * _CHUNK, _CHUNK)]],
                rows[c % _NBUF],
                gsem[c % _NBUF],
            )

        def write_chunk(c):
            return pltpu.async_copy(
                rows[c % _NBUF],
                out_hbm.at[pl.ds(base + c * _CHUNK, _CHUNK)],
                wsem[c % _NBUF],
            )

        gathers = {}
        writes = {}
        waited = set()
        for c in range(min(_NBUF - 1, n_chunks)):
            gathers[c] = gather_chunk(c)
        for c in range(n_chunks):
            gathers[c].wait()
            nxt = c + _NBUF - 1
            if nxt < n_chunks:
                prev = nxt - _NBUF
                if prev >= 0:
                    writes[prev].wait()
                    waited.add(prev)
                gathers[nxt] = gather_chunk(nxt)
            writes[c] = write_chunk(c)
        for c in range(n_chunks):
            if c not in waited:
                writes[c].wait()

    out = gather_kernel(emb_table, flat_idx)
    return out.reshape(_BATCH, _HIST, _DIM)
